# baseline (speedup 1.0000x reference)
import functools

import jax
import jax.numpy as jnp
from jax import lax
from jax.experimental import pallas as pl
from jax.experimental.pallas import tpu as pltpu
from jax.experimental.pallas import tpu_sc as plsc

NC, NS = 2, 16
B = 320000
D = 128
V = 10000
BPT = B // NS
C = 192
NCHUNK = 104
TAIL = BPT - NCHUNK * C
VSTAGE = 624


H = C // 2


def _gather_kernel(cell_hbm, hid_hbm, idx_hbm, out_cell, out_hid,
                   table_sh, idx0, idx1, rows0, rows1,
                   g0a, g0b, g1a, g1b, w0a, w0b, w1a, w1b, i0, i1):
    cid = lax.axis_index("c")
    sid = lax.axis_index("s")
    base = sid * BPT
    bufs = ((idx0, rows0, g0a, g0b, w0a, w0b, i0),
            (idx1, rows1, g1a, g1b, w1a, w1b, i1))

    def run_table(table_hbm, out_hbm):
        pltpu.async_copy(idx_hbm.at[pl.ds(pl.multiple_of(base, 8), C)],
                         idx0.at[pl.ds(0, C)], i0)
        pltpu.async_copy(idx_hbm.at[pl.ds(pl.multiple_of(base + C, 8), C)],
                         idx1.at[pl.ds(0, C)], i1)
        voff = pl.multiple_of(sid * VSTAGE, 8)
        pltpu.sync_copy(table_hbm.at[pl.ds(voff, VSTAGE)],
                        table_sh.at[pl.ds(voff, VSTAGE)])

        @pl.when(sid == NS - 1)
        def _():
            rem = NS * VSTAGE
            pltpu.sync_copy(table_hbm.at[pl.ds(rem, V - rem)],
                            table_sh.at[pl.ds(rem, V - rem)])

        plsc.subcore_barrier()

        def pf(off, n, b):
            idx_v = bufs[b][0]
            isem = bufs[b][6]
            pltpu.async_copy(idx_hbm.at[pl.ds(off, n)],
                             idx_v.at[pl.ds(0, n)], isem)

        def fire(off, n, b):
            idx_v, rows_v, gsa, gsb, _, _, isem = bufs[b]
            pltpu.make_async_copy(idx_hbm.at[pl.ds(off, n)],
                                  idx_v.at[pl.ds(0, n)], isem).wait()
            if n == C:
                pltpu.async_copy(table_sh.at[idx_v.at[pl.ds(0, H)]],
                                 rows_v.at[pl.ds(0, H)], gsa)
                pltpu.async_copy(table_sh.at[idx_v.at[pl.ds(H, H)]],
                                 rows_v.at[pl.ds(H, H)], gsb)
            else:
                pltpu.async_copy(table_sh.at[idx_v.at[pl.ds(0, n)]],
                                 rows_v.at[pl.ds(0, n)], gsa)

        def gwait_wstart(off, n, b):
            idx_v, rows_v, gsa, gsb, wsa, wsb, _ = bufs[b]
            if n == C:
                pltpu.make_async_copy(table_sh.at[idx_v.at[pl.ds(0, H)]],
                                      rows_v.at[pl.ds(0, H)], gsa).wait()
                pltpu.async_copy(rows_v.at[pl.ds(0, H)],
                                 out_hbm.at[pl.ds(off, H)], wsa)
                pltpu.make_async_copy(table_sh.at[idx_v.at[pl.ds(H, H)]],
                                      rows_v.at[pl.ds(H, H)], gsb).wait()
                pltpu.async_copy(rows_v.at[pl.ds(H, H)],
                                 out_hbm.at[pl.ds(pl.multiple_of(off + H, 8),
                                                  H)], wsb)
            else:
                pltpu.make_async_copy(table_sh.at[idx_v.at[pl.ds(0, n)]],
                                      rows_v.at[pl.ds(0, n)], gsa).wait()
                pltpu.async_copy(rows_v.at[pl.ds(0, n)],
                                 out_hbm.at[pl.ds(off, n)], wsa)

        def wwait(off, n, b):
            _, rows_v, _, _, wsa, wsb, _ = bufs[b]
            if n == C:
                pltpu.make_async_copy(rows_v.at[pl.ds(0, H)],
                                      out_hbm.at[pl.ds(off, H)], wsa).wait()
                pltpu.make_async_copy(rows_v.at[pl.ds(H, H)],
                                      out_hbm.at[pl.ds(pl.multiple_of(off + H,
                                                                      8),
                                                       H)], wsb).wait()
            else:
                pltpu.make_async_copy(rows_v.at[pl.ds(0, n)],
                                      out_hbm.at[pl.ds(off, n)], wsa).wait()

        def off_of(chunk):
            return pl.multiple_of(base + chunk * C, 8)

        fire(off_of(0), C, 0)
        fire(off_of(1), C, 1)
        gwait_wstart(off_of(0), C, 0)
        pf(off_of(2), C, 0)

        @pl.loop(0, NCHUNK - 4, step=2)
        def _(g):
            wwait(off_of(g), C, 0)
            fire(off_of(g + 2), C, 0)
            gwait_wstart(off_of(g + 1), C, 1)
            pf(off_of(g + 3), C, 1)
            wwait(off_of(g + 1), C, 1)
            fire(off_of(g + 3), C, 1)
            gwait_wstart(off_of(g + 2), C, 0)
            pf(off_of(g + 4), C, 0)

        tail_off = pl.multiple_of(base + NCHUNK * C, 8)
        wwait(off_of(NCHUNK - 4), C, 0)
        fire(off_of(NCHUNK - 2), C, 0)
        gwait_wstart(off_of(NCHUNK - 3), C, 1)
        pf(off_of(NCHUNK - 1), C, 1)
        wwait(off_of(NCHUNK - 3), C, 1)
        fire(off_of(NCHUNK - 1), C, 1)
        gwait_wstart(off_of(NCHUNK - 2), C, 0)
        pf(tail_off, TAIL, 0)
        wwait(off_of(NCHUNK - 2), C, 0)
        fire(tail_off, TAIL, 0)
        gwait_wstart(off_of(NCHUNK - 1), C, 1)
        gwait_wstart(tail_off, TAIL, 0)
        wwait(off_of(NCHUNK - 1), C, 1)
        wwait(tail_off, TAIL, 0)

    @pl.when(cid == 0)
    def _():
        run_table(cell_hbm, out_cell)

    @pl.when(cid == 1)
    def _():
        run_table(hid_hbm, out_hid)


def kernel(prev_cell, prev_hidden, child_indices):
    mesh = plsc.VectorSubcoreMesh(core_axis_name="c", subcore_axis_name="s")
    run = functools.partial(
        pl.kernel,
        out_type=(
            jax.ShapeDtypeStruct((B, D), jnp.float32),
            jax.ShapeDtypeStruct((B, D), jnp.float32),
        ),
        mesh=mesh,
        scratch_types=[
            pltpu.VMEM_SHARED((V, D), jnp.float32),
            pltpu.VMEM((C,), jnp.int32),
            pltpu.VMEM((C,), jnp.int32),
            pltpu.VMEM((C, D), jnp.float32),
            pltpu.VMEM((C, D), jnp.float32),
            pltpu.SemaphoreType.DMA,
            pltpu.SemaphoreType.DMA,
            pltpu.SemaphoreType.DMA,
            pltpu.SemaphoreType.DMA,
            pltpu.SemaphoreType.DMA,
            pltpu.SemaphoreType.DMA,
            pltpu.SemaphoreType.DMA,
            pltpu.SemaphoreType.DMA,
            pltpu.SemaphoreType.DMA,
            pltpu.SemaphoreType.DMA,
        ],
    )(_gather_kernel)
    return run(prev_cell, prev_hidden, child_indices.astype(jnp.int32))


# speedup vs baseline: 1.0121x; 1.0121x over previous
import functools

import jax
import jax.numpy as jnp
from jax import lax
from jax.experimental import pallas as pl
from jax.experimental.pallas import tpu as pltpu
from jax.experimental.pallas import tpu_sc as plsc

NC, NS = 2, 16
B = 320000
D = 128
V = 10000
BPT = B // NS
C = 192
NCHUNK = 104
TAIL = BPT - NCHUNK * C
VSTAGE = 624


H = C // 2


def _gather_kernel(cell_hbm, hid_hbm, idx_hbm, out_cell, out_hid,
                   table_sh, idx0, idx1, rows0, rows1,
                   g0a, g0b, g1a, g1b, w0, w1, ss, i0, i1):
    cid = lax.axis_index("c")
    sid = lax.axis_index("s")
    base = sid * BPT
    bufs = ((idx0, rows0, g0a, g0b, w0, i0),
            (idx1, rows1, g1a, g1b, w1, i1))

    def run_table(table_hbm, out_hbm):
        pltpu.async_copy(idx_hbm.at[pl.ds(pl.multiple_of(base, 8), C)],
                         idx0.at[pl.ds(0, C)], i0)
        pltpu.async_copy(idx_hbm.at[pl.ds(pl.multiple_of(base + C, 8), C)],
                         idx1.at[pl.ds(0, C)], i1)
        voff = pl.multiple_of(sid * VSTAGE, 8)
        rem = NS * VSTAGE
        pltpu.async_copy(table_hbm.at[pl.ds(voff, VSTAGE)],
                         table_sh.at[pl.ds(voff, VSTAGE)], ss)

        @pl.when(sid == NS - 1)
        def _():
            pltpu.async_copy(table_hbm.at[pl.ds(rem, V - rem)],
                             table_sh.at[pl.ds(rem, V - rem)], ss)

        def pf(off, n, b):
            idx_v = bufs[b][0]
            isem = bufs[b][5]
            pltpu.async_copy(idx_hbm.at[pl.ds(off, n)],
                             idx_v.at[pl.ds(0, n)], isem)

        def fire(off, n, b, src=None):
            idx_v, rows_v, gsa, gsb, _, isem = bufs[b]
            tbl = table_sh if src is None else src
            pltpu.make_async_copy(idx_hbm.at[pl.ds(off, n)],
                                  idx_v.at[pl.ds(0, n)], isem).wait()
            if n == C:
                pltpu.async_copy(tbl.at[idx_v.at[pl.ds(0, H)]],
                                 rows_v.at[pl.ds(0, H)], gsa)
                pltpu.async_copy(tbl.at[idx_v.at[pl.ds(H, H)]],
                                 rows_v.at[pl.ds(H, H)], gsb)
            else:
                pltpu.async_copy(tbl.at[idx_v.at[pl.ds(0, n)]],
                                 rows_v.at[pl.ds(0, n)], gsa)

        def gwait_wstart(off, n, b, src=None):
            idx_v, rows_v, gsa, gsb, wsem, _ = bufs[b]
            tbl = table_sh if src is None else src
            if n == C:
                pltpu.make_async_copy(tbl.at[idx_v.at[pl.ds(0, H)]],
                                      rows_v.at[pl.ds(0, H)], gsa).wait()
                pltpu.make_async_copy(tbl.at[idx_v.at[pl.ds(H, H)]],
                                      rows_v.at[pl.ds(H, H)], gsb).wait()
            else:
                pltpu.make_async_copy(tbl.at[idx_v.at[pl.ds(0, n)]],
                                      rows_v.at[pl.ds(0, n)], gsa).wait()
            pltpu.async_copy(rows_v.at[pl.ds(0, n)],
                             out_hbm.at[pl.ds(off, n)], wsem)

        def wwait(off, n, b):
            rows_v = bufs[b][1]
            wsem = bufs[b][4]
            pltpu.make_async_copy(rows_v.at[pl.ds(0, n)],
                                  out_hbm.at[pl.ds(off, n)], wsem).wait()

        def off_of(chunk):
            return pl.multiple_of(base + chunk * C, 8)

        fire(off_of(0), C, 0, src=table_hbm)
        fire(off_of(1), C, 1, src=table_hbm)
        gwait_wstart(off_of(0), C, 0, src=table_hbm)
        pf(off_of(2), C, 0)
        gwait_wstart(off_of(1), C, 1, src=table_hbm)
        pf(off_of(3), C, 1)
        pltpu.make_async_copy(table_hbm.at[pl.ds(voff, VSTAGE)],
                              table_sh.at[pl.ds(voff, VSTAGE)], ss).wait()

        @pl.when(sid == NS - 1)
        def _():
            pltpu.make_async_copy(table_hbm.at[pl.ds(rem, V - rem)],
                                  table_sh.at[pl.ds(rem, V - rem)], ss).wait()

        plsc.subcore_barrier()
        wwait(off_of(0), C, 0)
        fire(off_of(2), C, 0)
        wwait(off_of(1), C, 1)
        fire(off_of(3), C, 1)
        gwait_wstart(off_of(2), C, 0)
        pf(off_of(4), C, 0)

        @pl.loop(2, NCHUNK - 4, step=2)
        def _(g):
            wwait(off_of(g), C, 0)
            fire(off_of(g + 2), C, 0)
            gwait_wstart(off_of(g + 1), C, 1)
            pf(off_of(g + 3), C, 1)
            wwait(off_of(g + 1), C, 1)
            fire(off_of(g + 3), C, 1)
            gwait_wstart(off_of(g + 2), C, 0)
            pf(off_of(g + 4), C, 0)

        tail_off = pl.multiple_of(base + NCHUNK * C, 8)
        wwait(off_of(NCHUNK - 4), C, 0)
        fire(off_of(NCHUNK - 2), C, 0)
        gwait_wstart(off_of(NCHUNK - 3), C, 1)
        pf(off_of(NCHUNK - 1), C, 1)
        wwait(off_of(NCHUNK - 3), C, 1)
        fire(off_of(NCHUNK - 1), C, 1)
        gwait_wstart(off_of(NCHUNK - 2), C, 0)
        pf(tail_off, TAIL, 0)
        wwait(off_of(NCHUNK - 2), C, 0)
        fire(tail_off, TAIL, 0)
        gwait_wstart(off_of(NCHUNK - 1), C, 1)
        gwait_wstart(tail_off, TAIL, 0)
        wwait(off_of(NCHUNK - 1), C, 1)
        wwait(tail_off, TAIL, 0)

    @pl.when(cid == 0)
    def _():
        run_table(cell_hbm, out_cell)

    @pl.when(cid == 1)
    def _():
        run_table(hid_hbm, out_hid)


def kernel(prev_cell, prev_hidden, child_indices):
    mesh = plsc.VectorSubcoreMesh(core_axis_name="c", subcore_axis_name="s")
    run = functools.partial(
        pl.kernel,
        out_type=(
            jax.ShapeDtypeStruct((B, D), jnp.float32),
            jax.ShapeDtypeStruct((B, D), jnp.float32),
        ),
        mesh=mesh,
        scratch_types=[
            pltpu.VMEM_SHARED((V, D), jnp.float32),
            pltpu.VMEM((C,), jnp.int32),
            pltpu.VMEM((C,), jnp.int32),
            pltpu.VMEM((C, D), jnp.float32),
            pltpu.VMEM((C, D), jnp.float32),
            pltpu.SemaphoreType.DMA,
            pltpu.SemaphoreType.DMA,
            pltpu.SemaphoreType.DMA,
            pltpu.SemaphoreType.DMA,
            pltpu.SemaphoreType.DMA,
            pltpu.SemaphoreType.DMA,
            pltpu.SemaphoreType.DMA,
            pltpu.SemaphoreType.DMA,
            pltpu.SemaphoreType.DMA,
        ],
    )(_gather_kernel)
    return run(prev_cell, prev_hidden, child_indices.astype(jnp.int32))
